# baseline (speedup 1.0000x reference)
import functools

import jax
import jax.numpy as jnp
from jax import lax
from jax.experimental import pallas as pl
from jax.experimental.pallas import tpu as pltpu
from jax.experimental.pallas import tpu_sc as plsc

N = 10000
E = 320000
D_IN = 128
H1 = 64
H2 = 32
NUM_CLASSES = 8

NC = 2
NS = 16
NW = NC * NS
CHUNK = 128
CH = 80
CH2 = 2 * CH
EP = NW * CH * CHUNK
NP = 10240
RPT = NP // NS
HC = 16

_mesh = plsc.VectorSubcoreMesh(core_axis_name="c", subcore_axis_name="s")
_params = pltpu.CompilerParams(needs_layout_passes=False,
                               use_tc_tiling_on_sc=False)


def _deg_body(src_hbm, dst2_hbm, val_hbm, dego_out, degi_out,
              src_v, dst_v, val_v, zb1, deg_sh):
    cc = lax.axis_index("c")
    ss = lax.axis_index("s")
    w = cc * NS + ss
    pltpu.sync_copy(src_hbm.at[w], src_v)
    pltpu.sync_copy(dst2_hbm.at[w], dst_v)
    pltpu.sync_copy(val_hbm.at[w], val_v)

    def zb(r, carry):
        zb1[pl.ds(r * 16, 16)] = jnp.zeros((16,), jnp.float32)
        return carry
    lax.fori_loop(0, RPT // 16, zb, None)
    pltpu.sync_copy(zb1, deg_sh.at[pl.ds(ss * RPT, RPT)])
    pltpu.sync_copy(zb1, deg_sh.at[pl.ds(NP + ss * RPT, RPT)])
    plsc.subcore_barrier()

    def chunk(c, carry):
        pltpu.sync_copy(val_v.at[c], deg_sh.at[src_v.at[c]], add=True)
        pltpu.sync_copy(val_v.at[c], deg_sh.at[dst_v.at[c]], add=True)
        return carry
    lax.fori_loop(0, CH, chunk, None)
    plsc.subcore_barrier()
    sl = pl.ds(ss * RPT, RPT)
    pltpu.sync_copy(deg_sh.at[sl], dego_out.at[cc, sl])
    pltpu.sync_copy(deg_sh.at[pl.ds(NP + ss * RPT, RPT)], degi_out.at[cc, sl])


_deg_kernel = functools.partial(
    pl.kernel, _deg_body, mesh=_mesh, compiler_params=_params,
    out_type=(jax.ShapeDtypeStruct((NC, NP), jnp.float32),
              jax.ShapeDtypeStruct((NC, NP), jnp.float32)),
    scratch_types=[
        pltpu.VMEM((CH, CHUNK), jnp.int32),
        pltpu.VMEM((CH, CHUNK), jnp.int32),
        pltpu.VMEM((CH, CHUNK), jnp.float32),
        pltpu.VMEM((RPT,), jnp.float32),
        pltpu.VMEM_SHARED((2 * NP,), jnp.float32),
    ],
)()


def _edge_body(z_hbm, src_hbm, dst_hbm, ew_hbm, ns_hbm,
               s_out,
               src_v, dst_v, ew_v, w_v, ns_v, rows0, rows1, scb0, scb1, zbuf,
               acc_sh, z_sh, sem0, sem1, ssem0, ssem1):
    cc = lax.axis_index("c")
    ss = lax.axis_index("s")
    for k in range(2):
        blk = 2 * ss + k
        ksl = pl.ds(k * CH, CH)
        pltpu.sync_copy(src_hbm.at[blk], src_v.at[ksl])
        pltpu.sync_copy(dst_hbm.at[blk], dst_v.at[ksl])
        pltpu.sync_copy(ew_hbm.at[blk], ew_v.at[ksl])
    pltpu.sync_copy(ns_hbm, ns_v)
    zsl = pl.ds(ss * RPT, RPT)
    pltpu.sync_copy(z_hbm.at[cc, zsl], z_sh.at[zsl])
    _zero_rows(zbuf, acc_sh, ss)
    plsc.subcore_barrier()

    rows = (rows0, rows1)
    sems = (sem0, sem1)
    pltpu.async_copy(z_sh.at[src_v.at[0]], rows[0], sems[0])
    pltpu.async_copy(z_sh.at[src_v.at[1]], rows[1], sems[1])

    scbs = (scb0, scb1)
    ssems = (ssem0, ssem1)

    @pl.loop(0, CH2, step=2)
    def _chunks(c0):
        for b in range(2):
            c = c0 + b
            rows_v = rows[b]
            scb = scbs[b]
            for g in range(CHUNK // 16):
                sl = pl.ds(g * 16, 16)
                ns16 = plsc.load_gather(ns_v, [src_v[c, sl]])
                w_v[c, sl] = ew_v[c, sl] * ns16
            pltpu.make_async_copy(z_sh.at[src_v.at[c]], rows_v, sems[b]).wait()
            c16 = jnp.full((16,), c, jnp.int32)

            @plsc.parallel_loop(0, CHUNK, 1, unroll=16)
            def _scale(i):
                wspl = plsc.load_gather(w_v, [c16, jnp.full((16,), i, jnp.int32)])
                scb[i, pl.ds(0, HC)] = rows_v[i, pl.ds(0, HC)] * wspl

            @pl.when(c + 2 < CH2)
            def _():
                pltpu.async_copy(z_sh.at[src_v.at[c + 2]], rows_v, sems[b])

            @pl.when(c >= 2)
            def _():
                pltpu.make_async_copy(
                    scb, acc_sh.at[dst_v.at[c - 2]], ssems[b]).wait()
            pltpu.async_copy(scb, acc_sh.at[dst_v.at[c]], ssems[b], add=True)

    for b, cl in ((0, CH2 - 2), (1, CH2 - 1)):
        pltpu.make_async_copy(scbs[b], acc_sh.at[dst_v.at[cl]], ssems[b]).wait()
    plsc.subcore_barrier()
    sl = pl.ds(ss * RPT, RPT)
    pltpu.sync_copy(acc_sh.at[sl], s_out.at[cc, sl])


def _zero_rows(zbuf, shared, ss):
    def zb(r, carry):
        zbuf[r, pl.ds(0, 16)] = jnp.zeros((16,), jnp.float32)
        return carry
    lax.fori_loop(0, 128, zb, None)
    for k in range(RPT // 128):
        pltpu.sync_copy(zbuf, shared.at[pl.ds((ss * (RPT // 128) + k) * 128, 128)])


_edge_kernel = functools.partial(
    pl.kernel, _edge_body, mesh=_mesh, compiler_params=_params,
    out_type=jax.ShapeDtypeStruct((NC, NP, HC), jnp.float32),
    scratch_types=[
        pltpu.VMEM((CH2, CHUNK), jnp.int32),
        pltpu.VMEM((CH2, CHUNK), jnp.int32),
        pltpu.VMEM((CH2, CHUNK), jnp.float32),
        pltpu.VMEM((CH2, CHUNK), jnp.float32),
        pltpu.VMEM((NP,), jnp.float32),
        pltpu.VMEM((CHUNK, HC), jnp.float32),
        pltpu.VMEM((CHUNK, HC), jnp.float32),
        pltpu.VMEM((CHUNK, HC), jnp.float32),
        pltpu.VMEM((CHUNK, HC), jnp.float32),
        pltpu.VMEM((128, HC), jnp.float32),
        pltpu.VMEM_SHARED((NP, HC), jnp.float32),
        pltpu.VMEM_SHARED((NP, HC), jnp.float32),
        pltpu.SemaphoreType.DMA,
        pltpu.SemaphoreType.DMA,
        pltpu.SemaphoreType.DMA,
        pltpu.SemaphoreType.DMA,
    ],
)()


def _leaky(x):
    return jnp.where(x > 0, x, 0.01 * x)


def _dense1_body(x_ref, w1_ref, ewin_ref, val_ref, mu_ref, sg_ref,
                 dego_ref, degi_ref,
                 za_ref, zb_ref, ew_ref, ns_ref, nd_ref):
    mu = mu_ref[0, 0]
    sg = sg_ref[0, 0]
    ewr = ewin_ref[...]
    ew_ref[...] = jnp.where(ewr != 1.0, jnp.exp((ewr - mu) ** 2 / -sg),
                            ewr) * val_ref[...]
    dego = jnp.maximum(dego_ref[0] + dego_ref[1], 1.0)
    degi = jnp.maximum(degi_ref[0] + degi_ref[1], 1.0)
    ns_ref[...] = lax.rsqrt(dego)
    nd_ref[...] = lax.rsqrt(degi)
    for k, z_ref in enumerate((za_ref, zb_ref)):
        for c in range(NC):
            z_ref[c, pl.ds(0, N)] = jnp.dot(
                x_ref[...], w1_ref[:, pl.ds(c * 32 + k * HC, HC)],
                preferred_element_type=jnp.float32)
            z_ref[c, pl.ds(N, NP - N)] = jnp.zeros((NP - N, HC), jnp.float32)


_dense1 = pl.pallas_call(
    _dense1_body,
    out_shape=(jax.ShapeDtypeStruct((NC, NP, HC), jnp.float32),
               jax.ShapeDtypeStruct((NC, NP, HC), jnp.float32),
               jax.ShapeDtypeStruct((NW, CH, CHUNK), jnp.float32),
               jax.ShapeDtypeStruct((NP // 128, 128), jnp.float32),
               jax.ShapeDtypeStruct((NP // 128, 128), jnp.float32)),
)


def _dense2_body(sa_ref, sb_ref, nd_ref, b1_ref, w2_ref, z2_ref):
    s = jnp.concatenate([sa_ref[0], sb_ref[0], sa_ref[1], sb_ref[1]], axis=-1)
    h1 = _leaky(s * nd_ref[...] + b1_ref[...])
    for c in range(NC):
        z2_ref[c, pl.ds(0, N)] = jnp.dot(
            h1[:N],
            w2_ref[:, pl.ds(c * HC, HC)],
            preferred_element_type=jnp.float32)
        z2_ref[c, pl.ds(N, NP - N)] = jnp.zeros((NP - N, HC), jnp.float32)


_dense2 = pl.pallas_call(
    _dense2_body,
    out_shape=jax.ShapeDtypeStruct((NC, NP, HC), jnp.float32),
)


def _dense3_body(s_ref, nd_ref, b2_ref, wd1_ref, bd1_ref, wd2_ref, bd2_ref,
                 out_ref):
    s = jnp.concatenate([s_ref[0], s_ref[1]], axis=-1)
    h2 = _leaky(s * nd_ref[...] + b2_ref[...])
    hg = jnp.sum(h2[:N], axis=0, keepdims=True) * (1.0 / N)
    hg = _leaky(hg)
    r1 = jnp.sum(hg.reshape(H2, 1) * wd1_ref[...], axis=0, keepdims=True)
    r1 = _leaky(r1 + bd1_ref[...])
    r2 = jnp.sum(r1.reshape(16, 1) * wd2_ref[...], axis=0, keepdims=True)
    r2 = r2 + bd2_ref[...]
    out_ref[...] = 1.0 / (1.0 + jnp.exp(-r2))


_dense3 = pl.pallas_call(
    _dense3_body,
    out_shape=jax.ShapeDtypeStruct((1, NUM_CLASSES), jnp.float32),
)


def kernel(inputs, edge_index, edge_weight, W1, b1, W2, b2, Wd1, bd1, Wd2, bd2,
           mu, sigma):
    src = edge_index[0]
    dst = edge_index[1]
    pad = EP - E
    zi = jnp.zeros((pad,), jnp.int32)
    src_p = jnp.concatenate([src, zi]).reshape(NW, CH, CHUNK)
    dst_p = jnp.concatenate([dst, zi]).reshape(NW, CH, CHUNK)
    val_p = jnp.concatenate([jnp.ones((E,), jnp.float32),
                             jnp.zeros((pad,), jnp.float32)]).reshape(NW, CH, CHUNK)

    ewin_p = jnp.concatenate([edge_weight,
                              jnp.zeros((pad,), jnp.float32)]).reshape(NW, CH, CHUNK)

    dego_p, degi_p = _deg_kernel(src_p, dst_p + NP, val_p)

    za, zb, ew_p, ns2d, nd2d = _dense1(
        inputs, W1, ewin_p, val_p,
        mu.reshape(1, 1), sigma.reshape(1, 1),
        dego_p.reshape(NC, NP // 128, 128), degi_p.reshape(NC, NP // 128, 128))

    ns = ns2d.reshape(NP)
    nd_col = nd2d.reshape(NP, 1)

    s1a = _edge_kernel(za, src_p, dst_p, ew_p, ns)
    s1b = _edge_kernel(zb, src_p, dst_p, ew_p, ns)
    z2 = _dense2(s1a, s1b, nd_col, b1.reshape(1, H1), W2)
    s2 = _edge_kernel(z2, src_p, dst_p, ew_p, ns)
    out = _dense3(s2, nd_col, b2.reshape(1, H2), Wd1, bd1.reshape(1, 16),
                  Wd2, bd2.reshape(1, NUM_CLASSES))
    return out


# speedup vs baseline: 1.0232x; 1.0232x over previous
import functools

import jax
import jax.numpy as jnp
from jax import lax
from jax.experimental import pallas as pl
from jax.experimental.pallas import tpu as pltpu
from jax.experimental.pallas import tpu_sc as plsc

N = 10000
E = 320000
D_IN = 128
H1 = 64
H2 = 32
NUM_CLASSES = 8

NC = 2
NS = 16
NW = NC * NS
CHUNK = 128
CH = 80
CH2 = 2 * CH
EP = NW * CH * CHUNK
NP = 10240
RPT = NP // NS
HC = 16

_mesh = plsc.VectorSubcoreMesh(core_axis_name="c", subcore_axis_name="s")
_params = pltpu.CompilerParams(needs_layout_passes=False,
                               use_tc_tiling_on_sc=False)


def _deg_body(src_hbm, dst2_hbm, val_hbm, dego_out, degi_out,
              src_v, dst_v, val_v, zb1, deg_sh):
    cc = lax.axis_index("c")
    ss = lax.axis_index("s")
    w = cc * NS + ss
    pltpu.sync_copy(src_hbm.at[w], src_v)
    pltpu.sync_copy(dst2_hbm.at[w], dst_v)
    pltpu.sync_copy(val_hbm.at[w], val_v)

    def zb(r, carry):
        zb1[pl.ds(r * 16, 16)] = jnp.zeros((16,), jnp.float32)
        return carry
    lax.fori_loop(0, RPT // 16, zb, None)
    pltpu.sync_copy(zb1, deg_sh.at[pl.ds(ss * RPT, RPT)])
    pltpu.sync_copy(zb1, deg_sh.at[pl.ds(NP + ss * RPT, RPT)])
    plsc.subcore_barrier()

    def chunk(c, carry):
        pltpu.sync_copy(val_v.at[c], deg_sh.at[src_v.at[c]], add=True)
        pltpu.sync_copy(val_v.at[c], deg_sh.at[dst_v.at[c]], add=True)
        return carry
    lax.fori_loop(0, CH, chunk, None)
    plsc.subcore_barrier()
    sl = pl.ds(ss * RPT, RPT)
    pltpu.sync_copy(deg_sh.at[sl], dego_out.at[cc, sl])
    pltpu.sync_copy(deg_sh.at[pl.ds(NP + ss * RPT, RPT)], degi_out.at[cc, sl])


_deg_kernel = functools.partial(
    pl.kernel, _deg_body, mesh=_mesh, compiler_params=_params,
    out_type=(jax.ShapeDtypeStruct((NC, NP), jnp.float32),
              jax.ShapeDtypeStruct((NC, NP), jnp.float32)),
    scratch_types=[
        pltpu.VMEM((CH, CHUNK), jnp.int32),
        pltpu.VMEM((CH, CHUNK), jnp.int32),
        pltpu.VMEM((CH, CHUNK), jnp.float32),
        pltpu.VMEM((RPT,), jnp.float32),
        pltpu.VMEM_SHARED((2 * NP,), jnp.float32),
    ],
)()


def _edge_body(z_hbm, src_hbm, dst_hbm, ew_hbm, ns_hbm,
               s_out,
               src_v, dst_v, ew_v, w_v, ns_v, rows0, rows1, scb0, scb1, zbuf,
               acc_sh, z_sh, sem0, sem1, ssem0, ssem1):
    cc = lax.axis_index("c")
    ss = lax.axis_index("s")
    for k in range(2):
        blk = 2 * ss + k
        ksl = pl.ds(k * CH, CH)
        pltpu.sync_copy(src_hbm.at[blk], src_v.at[ksl])
        pltpu.sync_copy(dst_hbm.at[blk], dst_v.at[ksl])
        pltpu.sync_copy(ew_hbm.at[blk], ew_v.at[ksl])
    pltpu.sync_copy(ns_hbm, ns_v)
    zsl = pl.ds(ss * RPT, RPT)
    pltpu.sync_copy(z_hbm.at[cc, zsl], z_sh.at[zsl])
    _zero_rows(zbuf, acc_sh, ss)
    plsc.subcore_barrier()

    rows = (rows0, rows1)
    sems = (sem0, sem1)
    pltpu.async_copy(z_sh.at[src_v.at[0]], rows[0], sems[0])
    pltpu.async_copy(z_sh.at[src_v.at[1]], rows[1], sems[1])

    scbs = (scb0, scb1)
    ssems = (ssem0, ssem1)

    @pl.loop(0, CH2, step=2)
    def _chunks(c0):
        for b in range(2):
            c = c0 + b
            rows_v = rows[b]
            scb = scbs[b]
            for g in range(CHUNK // 16):
                sl = pl.ds(g * 16, 16)
                ns16 = plsc.load_gather(ns_v, [src_v[c, sl]])
                w_v[c, sl] = ew_v[c, sl] * ns16
            pltpu.make_async_copy(z_sh.at[src_v.at[c]], rows_v, sems[b]).wait()
            c16 = jnp.full((16,), c, jnp.int32)

            @plsc.parallel_loop(0, CHUNK, 1, unroll=8)
            def _scale(i):
                wspl = plsc.load_gather(w_v, [c16, jnp.full((16,), i, jnp.int32)])
                scb[i, pl.ds(0, HC)] = rows_v[i, pl.ds(0, HC)] * wspl

            @pl.when(c + 2 < CH2)
            def _():
                pltpu.async_copy(z_sh.at[src_v.at[c + 2]], rows_v, sems[b])

            @pl.when(c >= 2)
            def _():
                pltpu.make_async_copy(
                    scb, acc_sh.at[dst_v.at[c - 2]], ssems[b]).wait()
            pltpu.async_copy(scb, acc_sh.at[dst_v.at[c]], ssems[b], add=True)

    for b, cl in ((0, CH2 - 2), (1, CH2 - 1)):
        pltpu.make_async_copy(scbs[b], acc_sh.at[dst_v.at[cl]], ssems[b]).wait()
    plsc.subcore_barrier()
    sl = pl.ds(ss * RPT, RPT)
    pltpu.sync_copy(acc_sh.at[sl], s_out.at[cc, sl])


def _zero_rows(zbuf, shared, ss):
    def zb(r, carry):
        zbuf[r, pl.ds(0, 16)] = jnp.zeros((16,), jnp.float32)
        return carry
    lax.fori_loop(0, 128, zb, None)
    for k in range(RPT // 128):
        pltpu.sync_copy(zbuf, shared.at[pl.ds((ss * (RPT // 128) + k) * 128, 128)])


_edge_kernel = functools.partial(
    pl.kernel, _edge_body, mesh=_mesh, compiler_params=_params,
    out_type=jax.ShapeDtypeStruct((NC, NP, HC), jnp.float32),
    scratch_types=[
        pltpu.VMEM((CH2, CHUNK), jnp.int32),
        pltpu.VMEM((CH2, CHUNK), jnp.int32),
        pltpu.VMEM((CH2, CHUNK), jnp.float32),
        pltpu.VMEM((CH2, CHUNK), jnp.float32),
        pltpu.VMEM((NP,), jnp.float32),
        pltpu.VMEM((CHUNK, HC), jnp.float32),
        pltpu.VMEM((CHUNK, HC), jnp.float32),
        pltpu.VMEM((CHUNK, HC), jnp.float32),
        pltpu.VMEM((CHUNK, HC), jnp.float32),
        pltpu.VMEM((128, HC), jnp.float32),
        pltpu.VMEM_SHARED((NP, HC), jnp.float32),
        pltpu.VMEM_SHARED((NP, HC), jnp.float32),
        pltpu.SemaphoreType.DMA,
        pltpu.SemaphoreType.DMA,
        pltpu.SemaphoreType.DMA,
        pltpu.SemaphoreType.DMA,
    ],
)()


def _edge1_body(za_hbm, zb_hbm, src_hbm, dst_hbm, ew_hbm, ns_hbm,
                sa_out, sb_out,
                src_v, dst_v, ew_v, w_v, ns_v, rows0, rows1, scb0, scb1, zbuf,
                acc_sh, z_sh, sem0, sem1, ssem0, ssem1):
    cc = lax.axis_index("c")
    ss = lax.axis_index("s")
    for k in range(2):
        blk = 2 * ss + k
        ksl = pl.ds(k * CH, CH)
        pltpu.sync_copy(src_hbm.at[blk], src_v.at[ksl])
        pltpu.sync_copy(dst_hbm.at[blk], dst_v.at[ksl])
        pltpu.sync_copy(ew_hbm.at[blk], ew_v.at[ksl])
    pltpu.sync_copy(ns_hbm, ns_v)
    zsl = pl.ds(ss * RPT, RPT)
    rows = (rows0, rows1)
    sems = (sem0, sem1)
    scbs = (scb0, scb1)
    ssems = (ssem0, ssem1)

    for panel, (zp_hbm, sp_out, compute_w) in enumerate(
            ((za_hbm, sa_out, True), (zb_hbm, sb_out, False))):
        pltpu.sync_copy(zp_hbm.at[cc, zsl], z_sh.at[zsl])
        _zero_rows(zbuf, acc_sh, ss)
        plsc.subcore_barrier()

        pltpu.async_copy(z_sh.at[src_v.at[0]], rows[0], sems[0])
        pltpu.async_copy(z_sh.at[src_v.at[1]], rows[1], sems[1])

        @pl.loop(0, CH2, step=2)
        def _chunks(c0):
            for b in range(2):
                c = c0 + b
                rows_v = rows[b]
                scb = scbs[b]
                if compute_w:
                    for g in range(CHUNK // 16):
                        sl = pl.ds(g * 16, 16)
                        ns16 = plsc.load_gather(ns_v, [src_v[c, sl]])
                        w_v[c, sl] = ew_v[c, sl] * ns16
                pltpu.make_async_copy(z_sh.at[src_v.at[c]], rows_v,
                                      sems[b]).wait()
                c16 = jnp.full((16,), c, jnp.int32)

                @plsc.parallel_loop(0, CHUNK, 1, unroll=8)
                def _scale(i):
                    wspl = plsc.load_gather(
                        w_v, [c16, jnp.full((16,), i, jnp.int32)])
                    scb[i, pl.ds(0, HC)] = rows_v[i, pl.ds(0, HC)] * wspl

                @pl.when(c + 2 < CH2)
                def _():
                    pltpu.async_copy(z_sh.at[src_v.at[c + 2]], rows_v, sems[b])

                @pl.when(c >= 2)
                def _():
                    pltpu.make_async_copy(
                        scb, acc_sh.at[dst_v.at[c - 2]], ssems[b]).wait()
                pltpu.async_copy(scb, acc_sh.at[dst_v.at[c]], ssems[b],
                                 add=True)

        for b, cl in ((0, CH2 - 2), (1, CH2 - 1)):
            pltpu.make_async_copy(scbs[b], acc_sh.at[dst_v.at[cl]],
                                  ssems[b]).wait()
        plsc.subcore_barrier()
        sl = pl.ds(ss * RPT, RPT)
        pltpu.sync_copy(acc_sh.at[sl], sp_out.at[cc, sl])
        plsc.subcore_barrier()


_edge1_kernel = functools.partial(
    pl.kernel, _edge1_body, mesh=_mesh, compiler_params=_params,
    out_type=(jax.ShapeDtypeStruct((NC, NP, HC), jnp.float32),
              jax.ShapeDtypeStruct((NC, NP, HC), jnp.float32)),
    scratch_types=[
        pltpu.VMEM((CH2, CHUNK), jnp.int32),
        pltpu.VMEM((CH2, CHUNK), jnp.int32),
        pltpu.VMEM((CH2, CHUNK), jnp.float32),
        pltpu.VMEM((CH2, CHUNK), jnp.float32),
        pltpu.VMEM((NP,), jnp.float32),
        pltpu.VMEM((CHUNK, HC), jnp.float32),
        pltpu.VMEM((CHUNK, HC), jnp.float32),
        pltpu.VMEM((CHUNK, HC), jnp.float32),
        pltpu.VMEM((CHUNK, HC), jnp.float32),
        pltpu.VMEM((128, HC), jnp.float32),
        pltpu.VMEM_SHARED((NP, HC), jnp.float32),
        pltpu.VMEM_SHARED((NP, HC), jnp.float32),
        pltpu.SemaphoreType.DMA,
        pltpu.SemaphoreType.DMA,
        pltpu.SemaphoreType.DMA,
        pltpu.SemaphoreType.DMA,
    ],
)()

def _leaky(x):
    return jnp.where(x > 0, x, 0.01 * x)


def _dense1_body(x_ref, w1_ref, ewin_ref, val_ref, mu_ref, sg_ref,
                 dego_ref, degi_ref,
                 za_ref, zb_ref, ew_ref, ns_ref, nd_ref):
    mu = mu_ref[0, 0]
    sg = sg_ref[0, 0]
    ewr = ewin_ref[...]
    ew_ref[...] = jnp.where(ewr != 1.0, jnp.exp((ewr - mu) ** 2 / -sg),
                            ewr) * val_ref[...]
    dego = jnp.maximum(dego_ref[0] + dego_ref[1], 1.0)
    degi = jnp.maximum(degi_ref[0] + degi_ref[1], 1.0)
    ns_ref[...] = lax.rsqrt(dego)
    nd_ref[...] = lax.rsqrt(degi)
    for k, z_ref in enumerate((za_ref, zb_ref)):
        for c in range(NC):
            z_ref[c, pl.ds(0, N)] = jnp.dot(
                x_ref[...], w1_ref[:, pl.ds(c * 32 + k * HC, HC)],
                preferred_element_type=jnp.float32)
            z_ref[c, pl.ds(N, NP - N)] = jnp.zeros((NP - N, HC), jnp.float32)


_dense1 = pl.pallas_call(
    _dense1_body,
    out_shape=(jax.ShapeDtypeStruct((NC, NP, HC), jnp.float32),
               jax.ShapeDtypeStruct((NC, NP, HC), jnp.float32),
               jax.ShapeDtypeStruct((NW, CH, CHUNK), jnp.float32),
               jax.ShapeDtypeStruct((NP // 128, 128), jnp.float32),
               jax.ShapeDtypeStruct((NP // 128, 128), jnp.float32)),
)


def _dense2_body(sa_ref, sb_ref, nd_ref, b1_ref, w2_ref, z2_ref):
    s = jnp.concatenate([sa_ref[0], sb_ref[0], sa_ref[1], sb_ref[1]], axis=-1)
    h1 = _leaky(s * nd_ref[...] + b1_ref[...])
    for c in range(NC):
        z2_ref[c, pl.ds(0, N)] = jnp.dot(
            h1[:N],
            w2_ref[:, pl.ds(c * HC, HC)],
            preferred_element_type=jnp.float32)
        z2_ref[c, pl.ds(N, NP - N)] = jnp.zeros((NP - N, HC), jnp.float32)


_dense2 = pl.pallas_call(
    _dense2_body,
    out_shape=jax.ShapeDtypeStruct((NC, NP, HC), jnp.float32),
)


def _dense3_body(s_ref, nd_ref, b2_ref, wd1_ref, bd1_ref, wd2_ref, bd2_ref,
                 out_ref):
    s = jnp.concatenate([s_ref[0], s_ref[1]], axis=-1)
    h2 = _leaky(s * nd_ref[...] + b2_ref[...])
    hg = jnp.sum(h2[:N], axis=0, keepdims=True) * (1.0 / N)
    hg = _leaky(hg)
    r1 = jnp.sum(hg.reshape(H2, 1) * wd1_ref[...], axis=0, keepdims=True)
    r1 = _leaky(r1 + bd1_ref[...])
    r2 = jnp.sum(r1.reshape(16, 1) * wd2_ref[...], axis=0, keepdims=True)
    r2 = r2 + bd2_ref[...]
    out_ref[...] = 1.0 / (1.0 + jnp.exp(-r2))


_dense3 = pl.pallas_call(
    _dense3_body,
    out_shape=jax.ShapeDtypeStruct((1, NUM_CLASSES), jnp.float32),
)


def kernel(inputs, edge_index, edge_weight, W1, b1, W2, b2, Wd1, bd1, Wd2, bd2,
           mu, sigma):
    src = edge_index[0]
    dst = edge_index[1]
    pad = EP - E
    zi = jnp.zeros((pad,), jnp.int32)
    src_p = jnp.concatenate([src, zi]).reshape(NW, CH, CHUNK)
    dst_p = jnp.concatenate([dst, zi]).reshape(NW, CH, CHUNK)
    val_p = jnp.concatenate([jnp.ones((E,), jnp.float32),
                             jnp.zeros((pad,), jnp.float32)]).reshape(NW, CH, CHUNK)

    ewin_p = jnp.concatenate([edge_weight,
                              jnp.zeros((pad,), jnp.float32)]).reshape(NW, CH, CHUNK)

    dego_p, degi_p = _deg_kernel(src_p, dst_p + NP, val_p)

    za, zb, ew_p, ns2d, nd2d = _dense1(
        inputs, W1, ewin_p, val_p,
        mu.reshape(1, 1), sigma.reshape(1, 1),
        dego_p.reshape(NC, NP // 128, 128), degi_p.reshape(NC, NP // 128, 128))

    ns = ns2d.reshape(NP)
    nd_col = nd2d.reshape(NP, 1)

    s1a, s1b = _edge1_kernel(za, zb, src_p, dst_p, ew_p, ns)
    z2 = _dense2(s1a, s1b, nd_col, b1.reshape(1, H1), W2)
    s2 = _edge_kernel(z2, src_p, dst_p, ew_p, ns)
    out = _dense3(s2, nd_col, b2.reshape(1, H2), Wd1, bd1.reshape(1, 16),
                  Wd2, bd2.reshape(1, NUM_CLASSES))
    return out
